# baseline (speedup 1.0000x reference)
import jax
import jax.numpy as jnp
from jax.experimental import pallas as pl
from jax.experimental.pallas import tpu as pltpu

L = 8
VOCAB = 4
ED = 64
HID = 128
MP = 2
N = 256
EPM = 512
ST = 1.0
TT = 1.5
SR = 0.5
CP = 8
K = 4

def _flow_body(coords_ref, oh_ref, gsd_ref, gdiff_ref, gdstT_ref,
               a_emb_ref, wc_ref, beff_ref,
               wms_ref, wmd_ref, wmdist_ref, bmsg_ref,
               wuh_ref, wua_ref, bupd_ref,
               wo1_ref, bo1_ref, wsc_ref, bsc_ref, wsh_ref, bsh_ref,
               out_c_ref, out_ld_ref):
    oh = oh_ref[0]
    gsd = gsd_ref[...]
    gdiff = gdiff_ref[...]
    gdstT = gdstT_ref[...]
    parity = jax.lax.broadcasted_iota(jnp.int32, (N, 1), 0) % 2
    coords = [coords_ref[0, mol * N:(mol + 1) * N] for mol in range(K)]
    ohs = [oh[mol * N:(mol + 1) * N] for mol in range(K)]
    totals = [jnp.float32(0.0)] * K
    for l in range(L):
        active = (parity == (l % 2)).astype(jnp.float32)
        inactive = 1.0 - active
        hs, dists = [], []
        for mol in range(K):
            cc = coords[mol] * inactive
            h = ohs[mol] @ a_emb_ref[l] + cc @ wc_ref[l] + beff_ref[l]
            hs.append(jnp.maximum(h, 0.0))
            d = gdiff @ cc
            dists.append(jnp.sqrt(jnp.sum(d * d, axis=1, keepdims=True) + 1e-8))
        for m in range(MP):
            i = l * MP + m
            for mol in range(K):
                h = hs[mol]
                pre_s = h @ wms_ref[i]
                pre_d = h @ wmd_ref[i]
                pre = jnp.concatenate([pre_s, pre_d], axis=0)
                msg = gsd @ pre + dists[mol] * wmdist_ref[i] + bmsg_ref[i]
                msg = jnp.maximum(msg, 0.0)
                agg = gdstT @ msg
                hs[mol] = jnp.maximum(
                    h @ wuh_ref[i] + agg @ wua_ref[i] + bupd_ref[i], 0.0)
        for mol in range(K):
            h1 = jnp.maximum(hs[mol] @ wo1_ref[l] + bo1_ref[l], 0.0)
            raw_s = h1 @ wsc_ref[l] + bsc_ref[l]
            raw_sh = h1 @ wsh_ref[l] + bsh_ref[l]
            scale = SR * jnp.tanh(raw_s) * active
            coords[mol] = coords[mol] * jnp.exp(scale) + raw_sh * active
            totals[mol] = totals[mol] + jnp.sum(scale)
    out_c_ref[0] = jnp.concatenate(coords, axis=0)
    for mol in range(K):
        out_ld_ref[0, mol] = jnp.full((128,), totals[mol], jnp.float32)


def kernel(coordinates, atom_types, adj_list, atom_embed, W_in, b_in, W_msg,
           b_msg, W_upd, b_upd, W_o1, b_o1, W_o2, b_o2):
    f32 = jnp.float32
    Bn = coordinates.shape[0]
    G = Bn // K
    coords_p = jnp.pad(coordinates.astype(f32), ((0, 0), (0, 0), (0, CP - 3)))
    coords_p = coords_p.reshape(G, K * N, CP)
    oh = jax.nn.one_hot(atom_types, VOCAB, dtype=f32)
    oh = jnp.pad(oh, ((0, 0), (0, 0), (0, 8 - VOCAB))).reshape(G, K * N, 8)
    gsrc = jax.nn.one_hot(adj_list[:, 0], N, dtype=f32)
    gdst = jax.nn.one_hot(adj_list[:, 1], N, dtype=f32)
    gsd = jnp.concatenate([gsrc, gdst], axis=1)
    gdiff = gsrc - gdst
    gdstT = gdst.T

    a_emb = jnp.einsum('lve,leh->lvh', atom_embed, W_in[:, :ED])
    a_emb = jnp.pad(a_emb, ((0, 0), (0, 8 - VOCAB), (0, 0)))
    wc = jnp.pad(W_in[:, ED:ED + 3], ((0, 0), (0, CP - 3), (0, 0)))
    beff = (b_in + ST * W_in[:, ED + 3] + TT * W_in[:, ED + 4])[:, None]

    wms = W_msg[:, :, :HID].reshape(L * MP, HID, HID)
    wmd = W_msg[:, :, HID:2 * HID].reshape(L * MP, HID, HID)
    wmdist = W_msg[:, :, 2 * HID].reshape(L * MP, 1, HID)
    bmsg = b_msg.reshape(L * MP, 1, HID)
    wuh = W_upd[:, :, :HID].reshape(L * MP, HID, HID)
    wua = W_upd[:, :, HID:].reshape(L * MP, HID, HID)
    bupd = b_upd.reshape(L * MP, 1, HID)
    bo1 = b_o1[:, None]
    wsc = jnp.pad(W_o2[:, :, :3], ((0, 0), (0, 0), (0, CP - 3)))
    wsh = jnp.pad(W_o2[:, :, 3:6], ((0, 0), (0, 0), (0, CP - 3)))
    bsc = jnp.pad(b_o2[:, None, :3], ((0, 0), (0, 0), (0, CP - 3)))
    bsh = jnp.pad(b_o2[:, None, 3:6], ((0, 0), (0, 0), (0, CP - 3)))

    const = lambda *shape: pl.BlockSpec(shape, lambda b: (0,) * len(shape))
    grid_spec = pl.GridSpec(
        grid=(G,),
        in_specs=[
            pl.BlockSpec((1, K * N, CP), lambda b: (b, 0, 0)),
            pl.BlockSpec((1, K * N, 8), lambda b: (b, 0, 0)),
            const(EPM, 2 * N), const(EPM, N), const(N, EPM),
            const(L, 8, HID), const(L, CP, HID), const(L, 1, HID),
            const(L * MP, HID, HID), const(L * MP, HID, HID),
            const(L * MP, 1, HID), const(L * MP, 1, HID),
            const(L * MP, HID, HID), const(L * MP, HID, HID),
            const(L * MP, 1, HID),
            const(L, HID, HID), const(L, 1, HID),
            const(L, HID, CP), const(L, 1, CP),
            const(L, HID, CP), const(L, 1, CP),
        ],
        out_specs=[
            pl.BlockSpec((1, K * N, CP), lambda b: (b, 0, 0)),
            pl.BlockSpec((1, K, 128), lambda b: (b, 0, 0)),
        ],
    )
    out_c, out_ld = pl.pallas_call(
        _flow_body,
        grid_spec=grid_spec,
        out_shape=[
            jax.ShapeDtypeStruct((G, K * N, CP), f32),
            jax.ShapeDtypeStruct((G, K, 128), f32),
        ],
        compiler_params=pltpu.CompilerParams(
            dimension_semantics=("parallel",),
        ),
    )(coords_p, oh, gsd, gdiff, gdstT, a_emb, wc, beff,
      wms, wmd, wmdist, bmsg, wuh, wua, bupd,
      W_o1, bo1, wsc, bsc, wsh, bsh)
    return out_c.reshape(Bn, N, CP)[:, :, :3], out_ld.reshape(Bn, 128)[:, 0]


# speedup vs baseline: 1.7506x; 1.7506x over previous
import jax
import jax.numpy as jnp
from jax.experimental import pallas as pl
from jax.experimental.pallas import tpu as pltpu

L = 8
VOCAB = 4
ED = 64
HID = 128
MP = 2
N = 256
EPM = 512
ST = 1.0
TT = 1.5
SR = 0.5
CP = 8
K = 4

def _flow_body(coords_ref, oh_ref, gsd_ref, gdiff_ref, gdstT_ref,
               a_emb_ref, wc_ref, beff_ref,
               wms_ref, wmd_ref, wmdist_ref, bmsg_ref,
               wuh_ref, wua_ref, bupd_ref,
               wo1_ref, bo1_ref, wsc_ref, bsc_ref, wsh_ref, bsh_ref,
               out_c_ref, out_ld_ref):
    coords = coords_ref[0]
    oh = oh_ref[0]
    gsrc = gsd_ref[..., :N]
    gdst = gsd_ref[..., N:]
    gdiff = gdiff_ref[...]
    gdstT = gdstT_ref[...]
    parity = jax.lax.broadcasted_iota(jnp.int32, (K * N, 1), 0) % 2
    totals = [jnp.float32(0.0)] * K
    for l in range(L):
        active = (parity == (l % 2)).astype(jnp.float32)
        cc = coords * (1.0 - active)
        h = oh @ a_emb_ref[l] + cc @ wc_ref[l] + beff_ref[l]
        h = jnp.maximum(h, 0.0)
        cc_wide = jnp.concatenate(
            [cc[mol * N:(mol + 1) * N] for mol in range(K)], axis=1)
        d_wide = gdiff @ cc_wide
        dists = []
        for mol in range(K):
            d = d_wide[:, mol * CP:(mol + 1) * CP]
            dists.append(jnp.sqrt(jnp.sum(d * d, axis=1, keepdims=True) + 1e-8))
        for m in range(MP):
            i = l * MP + m
            pre_s = h @ wms_ref[i]
            pre_d = h @ wmd_ref[i]
            aggs = []
            for mol in range(K):
                sl = slice(mol * N, (mol + 1) * N)
                msg = (gsrc @ pre_s[sl] + gdst @ pre_d[sl]
                       + dists[mol] * wmdist_ref[i] + bmsg_ref[i])
                msg = jnp.maximum(msg, 0.0)
                aggs.append(gdstT @ msg)
            agg = jnp.concatenate(aggs, axis=0)
            h = jnp.maximum(h @ wuh_ref[i] + agg @ wua_ref[i] + bupd_ref[i], 0.0)
        h1 = jnp.maximum(h @ wo1_ref[l] + bo1_ref[l], 0.0)
        raw_s = h1 @ wsc_ref[l] + bsc_ref[l]
        raw_sh = h1 @ wsh_ref[l] + bsh_ref[l]
        scale = SR * jnp.tanh(raw_s) * active
        coords = coords * jnp.exp(scale) + raw_sh * active
        for mol in range(K):
            totals[mol] = totals[mol] + jnp.sum(scale[mol * N:(mol + 1) * N])
    out_c_ref[0] = coords
    for mol in range(K):
        out_ld_ref[0, mol] = jnp.full((128,), totals[mol], jnp.float32)


def kernel(coordinates, atom_types, adj_list, atom_embed, W_in, b_in, W_msg,
           b_msg, W_upd, b_upd, W_o1, b_o1, W_o2, b_o2):
    f32 = jnp.float32
    Bn = coordinates.shape[0]
    G = Bn // K
    coords_p = jnp.pad(coordinates.astype(f32), ((0, 0), (0, 0), (0, CP - 3)))
    coords_p = coords_p.reshape(G, K * N, CP)
    oh = jax.nn.one_hot(atom_types, VOCAB, dtype=f32)
    oh = jnp.pad(oh, ((0, 0), (0, 0), (0, 8 - VOCAB))).reshape(G, K * N, 8)
    gsrc = jax.nn.one_hot(adj_list[:, 0], N, dtype=f32)
    gdst = jax.nn.one_hot(adj_list[:, 1], N, dtype=f32)
    gsd = jnp.concatenate([gsrc, gdst], axis=1)
    gdiff = gsrc - gdst
    gdstT = gdst.T

    a_emb = jnp.einsum('lve,leh->lvh', atom_embed, W_in[:, :ED])
    a_emb = jnp.pad(a_emb, ((0, 0), (0, 8 - VOCAB), (0, 0)))
    wc = jnp.pad(W_in[:, ED:ED + 3], ((0, 0), (0, CP - 3), (0, 0)))
    beff = (b_in + ST * W_in[:, ED + 3] + TT * W_in[:, ED + 4])[:, None]

    wms = W_msg[:, :, :HID].reshape(L * MP, HID, HID)
    wmd = W_msg[:, :, HID:2 * HID].reshape(L * MP, HID, HID)
    wmdist = W_msg[:, :, 2 * HID].reshape(L * MP, 1, HID)
    bmsg = b_msg.reshape(L * MP, 1, HID)
    wuh = W_upd[:, :, :HID].reshape(L * MP, HID, HID)
    wua = W_upd[:, :, HID:].reshape(L * MP, HID, HID)
    bupd = b_upd.reshape(L * MP, 1, HID)
    bo1 = b_o1[:, None]
    wsc = jnp.pad(W_o2[:, :, :3], ((0, 0), (0, 0), (0, CP - 3)))
    wsh = jnp.pad(W_o2[:, :, 3:6], ((0, 0), (0, 0), (0, CP - 3)))
    bsc = jnp.pad(b_o2[:, None, :3], ((0, 0), (0, 0), (0, CP - 3)))
    bsh = jnp.pad(b_o2[:, None, 3:6], ((0, 0), (0, 0), (0, CP - 3)))

    const = lambda *shape: pl.BlockSpec(shape, lambda b: (0,) * len(shape))
    grid_spec = pl.GridSpec(
        grid=(G,),
        in_specs=[
            pl.BlockSpec((1, K * N, CP), lambda b: (b, 0, 0)),
            pl.BlockSpec((1, K * N, 8), lambda b: (b, 0, 0)),
            const(EPM, 2 * N), const(EPM, N), const(N, EPM),
            const(L, 8, HID), const(L, CP, HID), const(L, 1, HID),
            const(L * MP, HID, HID), const(L * MP, HID, HID),
            const(L * MP, 1, HID), const(L * MP, 1, HID),
            const(L * MP, HID, HID), const(L * MP, HID, HID),
            const(L * MP, 1, HID),
            const(L, HID, HID), const(L, 1, HID),
            const(L, HID, CP), const(L, 1, CP),
            const(L, HID, CP), const(L, 1, CP),
        ],
        out_specs=[
            pl.BlockSpec((1, K * N, CP), lambda b: (b, 0, 0)),
            pl.BlockSpec((1, K, 128), lambda b: (b, 0, 0)),
        ],
    )
    out_c, out_ld = pl.pallas_call(
        _flow_body,
        grid_spec=grid_spec,
        out_shape=[
            jax.ShapeDtypeStruct((G, K * N, CP), f32),
            jax.ShapeDtypeStruct((G, K, 128), f32),
        ],
        compiler_params=pltpu.CompilerParams(
            dimension_semantics=("parallel",),
        ),
    )(coords_p, oh, gsd, gdiff, gdstT, a_emb, wc, beff,
      wms, wmd, wmdist, bmsg, wuh, wua, bupd,
      W_o1, bo1, wsc, bsc, wsh, bsh)
    return out_c.reshape(Bn, N, CP)[:, :, :3], out_ld.reshape(Bn, 128)[:, 0]
